# baseline (speedup 1.0000x reference)
import functools

import jax
import jax.numpy as jnp
from jax import lax
from jax.experimental import pallas as pl
from jax.experimental.pallas import tpu as pltpu
from jax.experimental.pallas import tpu_sc as plsc

B = 16
SEQ = 2048
TOTAL = B * SEQ
H = 1024
V = 100000
L = 16

ROWS_PER_BLOCK = 1024


def _tw_body(hs_ref, w_ref, b_ref, out_ref):
    acc = jnp.dot(hs_ref[...], w_ref[...], preferred_element_type=jnp.float32)
    out_ref[...] = jnp.maximum(acc + b_ref[0, 0], 0.0)


def _token_weights(hidden_states, W, b):
    return pl.pallas_call(
        _tw_body,
        grid=(TOTAL // ROWS_PER_BLOCK,),
        in_specs=[
            pl.BlockSpec((ROWS_PER_BLOCK, H), lambda i: (i, 0)),
            pl.BlockSpec((H, 1), lambda i: (0, 0)),
            pl.BlockSpec(memory_space=pltpu.SMEM),
        ],
        out_specs=pl.BlockSpec((ROWS_PER_BLOCK, 1), lambda i: (i, 0)),
        out_shape=jax.ShapeDtypeStruct((TOTAL, 1), jnp.float32),
    )(hidden_states, W, b.reshape(1, 1))


def _sc_scatter_body(ids_hbm, tw_hbm, out_hbm, table_v, ids_v, tw_v):
    cid = lax.axis_index("c")
    sid = lax.axis_index("s")
    wid = sid * 2 + cid

    @pl.when(wid < B)
    def _():
        zeros = jnp.zeros((L,), jnp.float32)

        def zero_body(j, _):
            table_v[pl.ds(j * L, L)] = zeros
            return ()

        lax.fori_loop(0, V // L, zero_body, (), unroll=8)

        pltpu.sync_copy(ids_hbm.at[pl.ds(wid * SEQ, SEQ)], ids_v)
        pltpu.sync_copy(tw_hbm.at[pl.ds(wid * SEQ, SEQ)], tw_v)

        def tok_body(j, _):
            idx = ids_v[pl.ds(j * L, L)]
            w = tw_v[pl.ds(j * L, L)]
            cur = plsc.load_gather(table_v, [idx])

            def cond(cur):
                return jnp.any(w > cur)

            def body(cur):
                plsc.store_scatter(table_v, [idx], w, mask=w > cur)
                return plsc.load_gather(table_v, [idx])

            lax.while_loop(cond, body, cur)
            return ()

        lax.fori_loop(0, SEQ // L, tok_body, ())

        pltpu.sync_copy(table_v, out_hbm.at[pl.ds(wid * V, V)])


_sc_scatter = functools.partial(
    pl.kernel,
    out_type=jax.ShapeDtypeStruct((B * V,), jnp.float32),
    mesh=plsc.VectorSubcoreMesh(core_axis_name="c", subcore_axis_name="s"),
    compiler_params=pltpu.CompilerParams(needs_layout_passes=False),
    scratch_types=[
        pltpu.VMEM((V,), jnp.float32),
        pltpu.VMEM((SEQ,), jnp.int32),
        pltpu.VMEM((SEQ,), jnp.float32),
    ],
)(_sc_scatter_body)


@jax.jit
def kernel(hidden_states, extend_seq_lens, input_ids, W, b):
    del extend_seq_lens
    tw = _token_weights(hidden_states, W, b).reshape(TOTAL)
    ids = input_ids.astype(jnp.int32)
    flat = _sc_scatter(ids, tw)
    return flat.reshape(B, V)


# speedup vs baseline: 1.0691x; 1.0691x over previous
import functools

import jax
import jax.numpy as jnp
from jax import lax
from jax.experimental import pallas as pl
from jax.experimental.pallas import tpu as pltpu
from jax.experimental.pallas import tpu_sc as plsc

B = 16
SEQ = 2048
TOTAL = B * SEQ
H = 1024
V = 100000
L = 16

ROWS_PER_BLOCK = 4096


def _tw_body(hs_ref, w_ref, b_ref, out_ref):
    acc = jnp.dot(hs_ref[...], w_ref[...], preferred_element_type=jnp.float32)
    out_ref[...] = jnp.maximum(acc + b_ref[0, 0], 0.0)


def _token_weights(hidden_states, W, b):
    return pl.pallas_call(
        _tw_body,
        grid=(TOTAL // ROWS_PER_BLOCK,),
        in_specs=[
            pl.BlockSpec((ROWS_PER_BLOCK, H), lambda i: (i, 0)),
            pl.BlockSpec((H, 1), lambda i: (0, 0)),
            pl.BlockSpec(memory_space=pltpu.SMEM),
        ],
        out_specs=pl.BlockSpec((ROWS_PER_BLOCK, 1), lambda i: (i, 0)),
        out_shape=jax.ShapeDtypeStruct((TOTAL, 1), jnp.float32),
    )(hidden_states, W, b.reshape(1, 1))


def _sc_scatter_body(ids_hbm, tw_hbm, out_hbm, table_v, ids_v, tw_v):
    cid = lax.axis_index("c")
    sid = lax.axis_index("s")
    wid = sid * 2 + cid

    @pl.when(wid < B)
    def _():
        zeros = jnp.zeros((L,), jnp.float32)

        def zero_body(j, _):
            table_v[pl.ds(j * L, L)] = zeros
            return ()

        lax.fori_loop(0, V // L, zero_body, (), unroll=8)

        pltpu.sync_copy(ids_hbm.at[pl.ds(wid * SEQ, SEQ)], ids_v)
        pltpu.sync_copy(tw_hbm.at[pl.ds(wid * SEQ, SEQ)], tw_v)

        def tok_body(j, _):
            idx = ids_v[pl.ds(j * L, L)]
            w = tw_v[pl.ds(j * L, L)]
            cur = plsc.load_gather(table_v, [idx])

            def cond(cur):
                return jnp.any(w > cur)

            def body(cur):
                plsc.store_scatter(table_v, [idx], w, mask=w > cur)
                return plsc.load_gather(table_v, [idx])

            lax.while_loop(cond, body, cur)
            return ()

        lax.fori_loop(0, SEQ // L, tok_body, ())

        pltpu.sync_copy(table_v, out_hbm.at[pl.ds(wid * V, V)])


_sc_scatter = functools.partial(
    pl.kernel,
    out_type=jax.ShapeDtypeStruct((B * V,), jnp.float32),
    mesh=plsc.VectorSubcoreMesh(core_axis_name="c", subcore_axis_name="s"),
    compiler_params=pltpu.CompilerParams(needs_layout_passes=False),
    scratch_types=[
        pltpu.VMEM((V,), jnp.float32),
        pltpu.VMEM((SEQ,), jnp.int32),
        pltpu.VMEM((SEQ,), jnp.float32),
    ],
)(_sc_scatter_body)


@jax.jit
def kernel(hidden_states, extend_seq_lens, input_ids, W, b):
    del extend_seq_lens
    tw = _token_weights(hidden_states, W, b).reshape(TOTAL)
    ids = input_ids.astype(jnp.int32)
    flat = _sc_scatter(ids, tw)
    return flat.reshape(B, V)


# speedup vs baseline: 1.1036x; 1.0323x over previous
import functools

import jax
import jax.numpy as jnp
from jax import lax
from jax.experimental import pallas as pl
from jax.experimental.pallas import tpu as pltpu
from jax.experimental.pallas import tpu_sc as plsc

B = 16
SEQ = 2048
TOTAL = B * SEQ
H = 1024
V = 100000
L = 16

ROWS_PER_BLOCK = 2048


def _tw_body(hs_ref, w_ref, b_ref, out_ref):
    acc = jnp.dot(hs_ref[...], w_ref[...], preferred_element_type=jnp.float32)
    out_ref[...] = jnp.maximum(acc + b_ref[0, 0], 0.0)


def _token_weights(hidden_states, W, b):
    return pl.pallas_call(
        _tw_body,
        grid=(TOTAL // ROWS_PER_BLOCK,),
        in_specs=[
            pl.BlockSpec((ROWS_PER_BLOCK, H), lambda i: (i, 0)),
            pl.BlockSpec((H, 1), lambda i: (0, 0)),
            pl.BlockSpec(memory_space=pltpu.SMEM),
        ],
        out_specs=pl.BlockSpec((ROWS_PER_BLOCK, 1), lambda i: (i, 0)),
        out_shape=jax.ShapeDtypeStruct((TOTAL, 1), jnp.float32),
    )(hidden_states, W, b.reshape(1, 1))


HALF_V = V // 2


def _sc_scatter_body(ids_hbm, tw_hbm, out_hbm, table_v, ids_v, tw_v):
    cid = lax.axis_index("c")
    sid = lax.axis_index("s")
    wid = sid * 2 + cid
    batch = wid // 2
    lo = (wid % 2) * HALF_V

    zeros = jnp.zeros((L,), jnp.float32)

    def zero_body(j, _):
        table_v[pl.ds(j * L, L)] = zeros
        return ()

    lax.fori_loop(0, HALF_V // L, zero_body, (), unroll=8)

    pltpu.sync_copy(ids_hbm.at[pl.ds(batch * SEQ, SEQ)], ids_v)
    pltpu.sync_copy(tw_hbm.at[pl.ds(batch * SEQ, SEQ)], tw_v)

    def tok_body(j, _):
        idx = ids_v[pl.ds(j * L, L)] - lo
        w = tw_v[pl.ds(j * L, L)]
        in_r = (idx >= 0) & (idx < HALF_V)
        idx_c = jnp.clip(idx, 0, HALF_V - 1)
        cur = plsc.load_gather(table_v, [idx_c])

        def cond(cur):
            return jnp.any(in_r & (w > cur))

        def body(cur):
            plsc.store_scatter(table_v, [idx_c], w, mask=in_r & (w > cur))
            return plsc.load_gather(table_v, [idx_c])

        lax.while_loop(cond, body, cur)
        return ()

    lax.fori_loop(0, SEQ // L, tok_body, ())

    pltpu.sync_copy(table_v, out_hbm.at[pl.ds(batch * V + lo, HALF_V)])


_sc_scatter = functools.partial(
    pl.kernel,
    out_type=jax.ShapeDtypeStruct((B * V,), jnp.float32),
    mesh=plsc.VectorSubcoreMesh(core_axis_name="c", subcore_axis_name="s"),
    compiler_params=pltpu.CompilerParams(needs_layout_passes=False),
    scratch_types=[
        pltpu.VMEM((HALF_V,), jnp.float32),
        pltpu.VMEM((SEQ,), jnp.int32),
        pltpu.VMEM((SEQ,), jnp.float32),
    ],
)(_sc_scatter_body)


@jax.jit
def kernel(hidden_states, extend_seq_lens, input_ids, W, b):
    del extend_seq_lens
    tw = _token_weights(hidden_states, W, b).reshape(TOTAL)
    ids = input_ids.astype(jnp.int32)
    flat = _sc_scatter(ids, tw)
    return flat.reshape(B, V)
